# Initial kernel scaffold; baseline (speedup 1.0000x reference)
import jax
import jax.numpy as jnp
from jax.experimental import pallas as pl


def kernel(input):
    raise NotImplementedError("write your pallas kernel here")


# speedup vs baseline: 8.4452x; 8.4452x over previous
import functools

import jax
import jax.numpy as jnp
from jax import lax
from jax.experimental import pallas as pl
from jax.experimental.pallas import tpu as pltpu
from jax.experimental.pallas import tpu_sc as plsc

ROWS = 64
N = 8192
L = 16
NC = 2
NS = 16
NW = NC * NS
ROWS_PER_W = ROWS // NW
NCHUNK = N // L
BISECT_ITERS = 26

_f32 = jnp.float32


@functools.partial(
    pl.kernel,
    out_type=jax.ShapeDtypeStruct((ROWS, N), _f32),
    mesh=plsc.VectorSubcoreMesh(core_axis_name="c", subcore_axis_name="s"),
    scratch_types=[
        pltpu.VMEM((ROWS_PER_W, N), _f32),
        pltpu.VMEM((N + L,), _f32),
    ],
    compiler_params=pltpu.CompilerParams(needs_layout_passes=False),
)
def _sparsemax_sc(x_hbm, out_hbm, xv, cand):
    wid = lax.axis_index("s") * NC + lax.axis_index("c")
    base = wid * ROWS_PER_W
    pltpu.sync_copy(x_hbm.at[pl.ds(base, ROWS_PER_W)], xv)

    for r in range(ROWS_PER_W):
        def mx_body(i, acc):
            return jnp.maximum(acc, xv[r, pl.ds(i * L, L)])

        acc = lax.fori_loop(0, NCHUNK, mx_body, jnp.full((L,), -jnp.inf, _f32))
        rowmax = jnp.max(acc)
        t0 = rowmax - _f32(1.0)

        def cp_body(i, off):
            v = xv[r, pl.ds(i * L, L)]
            m = v > t0
            plsc.store_compressed(cand.at[pl.ds(off, L)], v, mask=m)
            return off + jnp.sum(m.astype(jnp.int32))

        mcount = lax.fori_loop(0, NCHUNK, cp_body, jnp.int32(0))
        cand[pl.ds(mcount, L)] = jnp.full((L,), t0, _f32)
        nchunks = (mcount + (L - 1)) >> 4

        def bis_body(_, lohi):
            lo, hi = lohi
            mid = _f32(0.5) * (lo + hi)

            def s_body(i, sacc):
                v = cand[pl.ds(i * L, L)]
                return sacc + jnp.maximum(v - mid, _f32(0.0))

            s = jnp.sum(lax.fori_loop(0, nchunks, s_body,
                                      jnp.zeros((L,), _f32)))
            ge = s >= _f32(1.0)
            return (jnp.where(ge, mid, lo), jnp.where(ge, hi, mid))

        lo, hi = lax.fori_loop(0, BISECT_ITERS, bis_body, (t0, rowmax))

        def fin_body(i, carry):
            sacc, cacc = carry
            v = cand[pl.ds(i * L, L)]
            m = v > lo
            return (sacc + jnp.where(m, v, _f32(0.0)),
                    cacc + m.astype(_f32))

        sacc, cacc = lax.fori_loop(
            0, nchunks, fin_body,
            (jnp.zeros((L,), _f32), jnp.zeros((L,), _f32)))
        sv = jnp.full((L,), jnp.sum(sacc) - _f32(1.0), _f32)
        cv = jnp.full((L,), jnp.maximum(jnp.sum(cacc), _f32(1.0)), _f32)
        tau = sv / cv

        def out_body(i, carry):
            sl = pl.ds(i * L, L)
            xv[r, sl] = jnp.maximum(xv[r, sl] - tau, _f32(0.0))
            return carry

        lax.fori_loop(0, NCHUNK, out_body, jnp.int32(0))

    pltpu.sync_copy(xv, out_hbm.at[pl.ds(base, ROWS_PER_W)])


def kernel(input):
    return _sparsemax_sc(input)


# speedup vs baseline: 9.8585x; 1.1674x over previous
import functools

import jax
import jax.numpy as jnp
from jax import lax
from jax.experimental import pallas as pl
from jax.experimental.pallas import tpu as pltpu
from jax.experimental.pallas import tpu_sc as plsc

ROWS = 64
N = 8192
L = 16
NC = 2
NS = 16
NW = NC * NS
ROWS_PER_W = ROWS // NW
NCHUNK = N // L
BISECT_ITERS = 26

_f32 = jnp.float32


@functools.partial(
    pl.kernel,
    out_type=jax.ShapeDtypeStruct((ROWS, N), _f32),
    mesh=plsc.VectorSubcoreMesh(core_axis_name="c", subcore_axis_name="s"),
    scratch_types=[
        pltpu.VMEM((ROWS_PER_W, N), _f32),
        pltpu.VMEM((N + L,), _f32),
    ],
    compiler_params=pltpu.CompilerParams(needs_layout_passes=False),
)
def _sparsemax_sc(x_hbm, out_hbm, xv, cand):
    wid = lax.axis_index("s") * NC + lax.axis_index("c")
    base = wid * ROWS_PER_W
    pltpu.sync_copy(x_hbm.at[pl.ds(base, ROWS_PER_W)], xv)

    for r in range(ROWS_PER_W):
        neg = jnp.full((L,), -jnp.inf, _f32)

        @plsc.parallel_loop(0, NCHUNK, step=4, unroll=2,
                            carry=(neg, neg, neg, neg))
        def mx_accs(i, accs):
            return tuple(
                jnp.maximum(a, xv[r, pl.ds((i + j) * L, L)])
                for j, a in enumerate(accs))

        a0, a1, a2, a3 = mx_accs
        rowmax = jnp.max(jnp.maximum(jnp.maximum(a0, a1),
                                     jnp.maximum(a2, a3)))
        t0 = rowmax - _f32(1.0)

        def cp_body(i, off):
            for j in range(4):
                v = xv[r, pl.ds((i * 4 + j) * L, L)]
                m = v > t0
                plsc.store_compressed(cand.at[pl.ds(off, L)], v, mask=m)
                off = off + jnp.sum(m.astype(jnp.int32))
            return off

        mcount = lax.fori_loop(0, NCHUNK // 4, cp_body, jnp.int32(0))
        cand[pl.ds(mcount, L)] = jnp.full((L,), t0, _f32)
        nchunks = (mcount + (L - 1)) >> 4

        def bis_body(_, lohi):
            lo, hi = lohi
            mid = _f32(0.5) * (lo + hi)

            def s_body(i, sacc):
                v = cand[pl.ds(i * L, L)]
                return sacc + jnp.maximum(v - mid, _f32(0.0))

            s = jnp.sum(lax.fori_loop(0, nchunks, s_body,
                                      jnp.zeros((L,), _f32)))
            ge = s >= _f32(1.0)
            return (jnp.where(ge, mid, lo), jnp.where(ge, hi, mid))

        lo, hi = lax.fori_loop(0, BISECT_ITERS, bis_body, (t0, rowmax))

        def fin_body(i, carry):
            sacc, cacc = carry
            v = cand[pl.ds(i * L, L)]
            m = v > lo
            return (sacc + jnp.where(m, v, _f32(0.0)),
                    cacc + m.astype(_f32))

        sacc, cacc = lax.fori_loop(
            0, nchunks, fin_body,
            (jnp.zeros((L,), _f32), jnp.zeros((L,), _f32)))
        sv = jnp.full((L,), jnp.sum(sacc) - _f32(1.0), _f32)
        cv = jnp.full((L,), jnp.maximum(jnp.sum(cacc), _f32(1.0)), _f32)
        tau = sv / cv

        @plsc.parallel_loop(0, NCHUNK, step=4, unroll=2)
        def _out(i):
            for j in range(4):
                sl = pl.ds((i + j) * L, L)
                xv[r, sl] = jnp.maximum(xv[r, sl] - tau, _f32(0.0))

    pltpu.sync_copy(xv, out_hbm.at[pl.ds(base, ROWS_PER_W)])


def kernel(input):
    return _sparsemax_sc(input)


# speedup vs baseline: 10.3661x; 1.0515x over previous
import functools

import jax
import jax.numpy as jnp
from jax import lax
from jax.experimental import pallas as pl
from jax.experimental.pallas import tpu as pltpu
from jax.experimental.pallas import tpu_sc as plsc

ROWS = 64
N = 8192
L = 16
NC = 2
NS = 16
NW = NC * NS
ROWS_PER_W = ROWS // NW
NCHUNK = N // L
BISECT_ITERS = 26

_f32 = jnp.float32


@functools.partial(
    pl.kernel,
    out_type=jax.ShapeDtypeStruct((ROWS, N), _f32),
    mesh=plsc.VectorSubcoreMesh(core_axis_name="c", subcore_axis_name="s"),
    scratch_types=[
        pltpu.VMEM((ROWS_PER_W, N), _f32),
        pltpu.VMEM((N + L,), _f32),
    ],
    compiler_params=pltpu.CompilerParams(needs_layout_passes=False),
)
def _sparsemax_sc(x_hbm, out_hbm, xv, cand):
    wid = lax.axis_index("s") * NC + lax.axis_index("c")
    base = wid * ROWS_PER_W
    pltpu.sync_copy(x_hbm.at[pl.ds(base, ROWS_PER_W)], xv)

    for r in range(ROWS_PER_W):
        neg = jnp.full((L,), -jnp.inf, _f32)

        @plsc.parallel_loop(0, NCHUNK, step=4, unroll=2,
                            carry=(neg, neg, neg, neg))
        def mx_accs(i, accs):
            return tuple(
                jnp.maximum(a, xv[r, pl.ds((i + j) * L, L)])
                for j, a in enumerate(accs))

        a0, a1, a2, a3 = mx_accs
        rowmax = jnp.max(jnp.maximum(jnp.maximum(a0, a1),
                                     jnp.maximum(a2, a3)))
        t0 = rowmax - _f32(1.0)

        def cp_body(i, off):
            for j in range(4):
                v = xv[r, pl.ds((i * 4 + j) * L, L)]
                m = v > t0
                plsc.store_compressed(cand.at[pl.ds(off, L)], v, mask=m)
                off = off + plsc.all_reduce_population_count(m)[0]
            return off

        mcount = lax.fori_loop(0, NCHUNK // 4, cp_body, jnp.int32(0))
        cand[pl.ds(mcount, L)] = jnp.full((L,), t0, _f32)
        nchunks = (mcount + (L - 1)) >> 4

        def bis_body(_, lohi):
            lo, hi = lohi
            mid = _f32(0.5) * (lo + hi)

            def s_body(i, sacc):
                v = cand[pl.ds(i * L, L)]
                return sacc + jnp.maximum(v - mid, _f32(0.0))

            s = jnp.sum(lax.fori_loop(0, nchunks, s_body,
                                      jnp.zeros((L,), _f32)))
            ge = s >= _f32(1.0)
            return (jnp.where(ge, mid, lo), jnp.where(ge, hi, mid))

        lo, hi = lax.fori_loop(0, BISECT_ITERS, bis_body, (t0, rowmax))

        def fin_body(i, carry):
            sacc, cacc = carry
            v = cand[pl.ds(i * L, L)]
            m = v > lo
            return (sacc + jnp.where(m, v, _f32(0.0)),
                    cacc + m.astype(_f32))

        sacc, cacc = lax.fori_loop(
            0, nchunks, fin_body,
            (jnp.zeros((L,), _f32), jnp.zeros((L,), _f32)))
        sv = jnp.full((L,), jnp.sum(sacc) - _f32(1.0), _f32)
        cv = jnp.full((L,), jnp.maximum(jnp.sum(cacc), _f32(1.0)), _f32)
        tau = sv / cv

        @plsc.parallel_loop(0, NCHUNK, step=4, unroll=2)
        def _out(i):
            for j in range(4):
                sl = pl.ds((i + j) * L, L)
                xv[r, sl] = jnp.maximum(xv[r, sl] - tau, _f32(0.0))

    pltpu.sync_copy(xv, out_hbm.at[pl.ds(base, ROWS_PER_W)])


def kernel(input):
    return _sparsemax_sc(input)


# speedup vs baseline: 12.5189x; 1.2077x over previous
import functools

import jax
import jax.numpy as jnp
from jax import lax
from jax.experimental import pallas as pl
from jax.experimental.pallas import tpu as pltpu
from jax.experimental.pallas import tpu_sc as plsc

ROWS = 64
N = 8192
L = 16
NC = 2
NS = 16
NW = NC * NS
ROWS_PER_W = ROWS // NW
NCHUNK = N // L
NSEG = 4
SEG = N // NSEG + L
BISECT_ITERS = 26

_f32 = jnp.float32


@functools.partial(
    pl.kernel,
    out_type=jax.ShapeDtypeStruct((ROWS, N), _f32),
    mesh=plsc.VectorSubcoreMesh(core_axis_name="c", subcore_axis_name="s"),
    scratch_types=[
        pltpu.VMEM((ROWS_PER_W, N), _f32),
        pltpu.VMEM((NSEG * SEG,), _f32),
        pltpu.VMEM((N + 6 * L,), _f32),
        pltpu.VMEM((NSEG * SEG,), jnp.int32),
        pltpu.VMEM((N + 6 * L,), jnp.int32),
    ],
    compiler_params=pltpu.CompilerParams(needs_layout_passes=False),
)
def _sparsemax_sc(x_hbm, out_hbm, xv, cand, packed, cidx, pidx):
    wid = lax.axis_index("s") * NC + lax.axis_index("c")
    base = wid * ROWS_PER_W
    pltpu.sync_copy(x_hbm.at[pl.ds(base, ROWS_PER_W)], xv)

    for r in range(ROWS_PER_W):
        neg = jnp.full((L,), -jnp.inf, _f32)

        @plsc.parallel_loop(0, NCHUNK, step=4, unroll=2,
                            carry=(neg, neg, neg, neg))
        def mx_accs(i, accs):
            return tuple(
                jnp.maximum(a, xv[r, pl.ds((i + j) * L, L)])
                for j, a in enumerate(accs))

        a0, a1, a2, a3 = mx_accs
        rowmax = jnp.max(jnp.maximum(jnp.maximum(a0, a1),
                                     jnp.maximum(a2, a3)))
        t0 = rowmax - _f32(1.0)

        z32 = jnp.int32(0)

        lane = lax.iota(jnp.int32, L)

        @plsc.parallel_loop(0, NCHUNK, step=NSEG, unroll=2,
                            carry=(z32, z32, z32, z32))
        def cp_offs(i, offs):
            nxt = []
            for j, off in enumerate(offs):
                v = xv[r, pl.ds((i + j) * L, L)]
                m = v > t0
                plsc.store_compressed(cand.at[pl.ds(j * SEG + off, L)],
                                      v, mask=m)
                plsc.store_compressed(cidx.at[pl.ds(j * SEG + off, L)],
                                      lane + (i + j) * L, mask=m)
                nxt.append(off + plsc.all_reduce_population_count(m)[0])
            return tuple(nxt)

        tpad = jnp.full((L,), t0, _f32)
        zpad = jnp.zeros((L,), jnp.int32)
        woff = z32
        for j in range(NSEG):
            oj = cp_offs[j]
            cand[pl.ds(j * SEG + oj, L)] = tpad
            cidx[pl.ds(j * SEG + oj, L)] = zpad
            ncj = (oj + (L - 1)) >> 4

            def pk_body(k, w, j=j):
                packed[pl.ds(w * L, L)] = cand[pl.ds(j * SEG + k * L, L)]
                pidx[pl.ds(w * L, L)] = cidx[pl.ds(j * SEG + k * L, L)]
                return w + 1

            woff = lax.fori_loop(0, ncj, pk_body, woff)
        packed[pl.ds(woff * L, L)] = tpad
        pidx[pl.ds(woff * L, L)] = zpad
        trips = (woff + 1) >> 1

        def bis_body(_, lohi):
            lo, hi = lohi
            mid = _f32(0.5) * (lo + hi)

            def s_body(k, accs):
                s0, s1 = accs
                v0 = packed[pl.ds((2 * k) * L, L)]
                v1 = packed[pl.ds((2 * k + 1) * L, L)]
                return (s0 + jnp.maximum(v0 - mid, _f32(0.0)),
                        s1 + jnp.maximum(v1 - mid, _f32(0.0)))

            zv = jnp.zeros((L,), _f32)
            s0, s1 = lax.fori_loop(0, trips, s_body, (zv, zv))
            s = jnp.sum(s0 + s1)
            ge = s >= _f32(1.0)
            return (jnp.where(ge, mid, lo), jnp.where(ge, hi, mid))

        lo, hi = lax.fori_loop(0, BISECT_ITERS, bis_body, (t0, rowmax))

        def fin_body(k, carry):
            sacc, cacc = carry
            v0 = packed[pl.ds((2 * k) * L, L)]
            v1 = packed[pl.ds((2 * k + 1) * L, L)]
            m0 = v0 > lo
            m1 = v1 > lo
            return (sacc + jnp.where(m0, v0, _f32(0.0))
                    + jnp.where(m1, v1, _f32(0.0)),
                    cacc + m0.astype(_f32) + m1.astype(_f32))

        sacc, cacc = lax.fori_loop(
            0, trips, fin_body,
            (jnp.zeros((L,), _f32), jnp.zeros((L,), _f32)))
        sv = jnp.full((L,), jnp.sum(sacc) - _f32(1.0), _f32)
        cv = jnp.full((L,), jnp.maximum(jnp.sum(cacc), _f32(1.0)), _f32)
        tau = sv / cv

        zf = jnp.zeros((L,), _f32)

        @plsc.parallel_loop(0, NCHUNK, step=4, unroll=2)
        def _zero(i):
            for j in range(4):
                xv[r, pl.ds((i + j) * L, L)] = zf

        rvec = jnp.full((L,), r, jnp.int32)

        def sct_body(k, c):
            v0 = packed[pl.ds((2 * k) * L, L)]
            v1 = packed[pl.ds((2 * k + 1) * L, L)]
            i0 = pidx[pl.ds((2 * k) * L, L)]
            i1 = pidx[pl.ds((2 * k + 1) * L, L)]
            plsc.store_scatter(xv, [rvec, i0],
                               jnp.maximum(v0 - tau, _f32(0.0)),
                               mask=v0 > lo)
            plsc.store_scatter(xv, [rvec, i1],
                               jnp.maximum(v1 - tau, _f32(0.0)),
                               mask=v1 > lo)
            return c

        lax.fori_loop(0, trips, sct_body, z32)

    pltpu.sync_copy(xv, out_hbm.at[pl.ds(base, ROWS_PER_W)])


def kernel(input):
    return _sparsemax_sc(input)


# speedup vs baseline: 13.0977x; 1.0462x over previous
import functools

import jax
import jax.numpy as jnp
from jax import lax
from jax.experimental import pallas as pl
from jax.experimental.pallas import tpu as pltpu
from jax.experimental.pallas import tpu_sc as plsc

ROWS = 64
N = 8192
L = 16
NC = 2
NS = 16
NW = NC * NS
ROWS_PER_W = ROWS // NW
NCHUNK = N // L
NSEG = 4
SEG = N // NSEG + L
BISECT_ITERS = 26

_f32 = jnp.float32


@functools.partial(
    pl.kernel,
    out_type=jax.ShapeDtypeStruct((ROWS, N), _f32),
    mesh=plsc.VectorSubcoreMesh(core_axis_name="c", subcore_axis_name="s"),
    scratch_types=[
        pltpu.VMEM((ROWS_PER_W, N), _f32),
        pltpu.VMEM((NSEG * SEG,), _f32),
        pltpu.VMEM((N + 6 * L,), _f32),
        pltpu.SemaphoreType.DMA,
        pltpu.SemaphoreType.DMA,
    ],
    compiler_params=pltpu.CompilerParams(needs_layout_passes=False),
)
def _sparsemax_sc(x_hbm, out_hbm, xv, cand, packed, sem_in1, sem_out0):
    wid = lax.axis_index("s") * NC + lax.axis_index("c")
    base = wid * ROWS_PER_W

    in1 = pltpu.async_copy(x_hbm.at[pl.ds(base + 1, 1)],
                           xv.at[pl.ds(1, 1)], sem_in1)
    pltpu.sync_copy(x_hbm.at[pl.ds(base, 1)], xv.at[pl.ds(0, 1)])

    for r in range(ROWS_PER_W):
        if r == 1:
            in1.wait()
        neg = jnp.full((L,), -jnp.inf, _f32)

        @plsc.parallel_loop(0, NCHUNK, step=4, unroll=2,
                            carry=(neg, neg, neg, neg))
        def mx_accs(i, accs):
            return tuple(
                jnp.maximum(a, xv[r, pl.ds((i + j) * L, L)])
                for j, a in enumerate(accs))

        a0, a1, a2, a3 = mx_accs
        rowmax = jnp.max(jnp.maximum(jnp.maximum(a0, a1),
                                     jnp.maximum(a2, a3)))
        t0 = rowmax - _f32(1.0)

        z32 = jnp.int32(0)

        @plsc.parallel_loop(0, NCHUNK, step=NSEG, unroll=2,
                            carry=(z32, z32, z32, z32))
        def cp_offs(i, offs):
            nxt = []
            for j, off in enumerate(offs):
                v = xv[r, pl.ds((i + j) * L, L)]
                m = v > t0
                plsc.store_compressed(cand.at[pl.ds(j * SEG + off, L)],
                                      v, mask=m)
                nxt.append(off + plsc.all_reduce_population_count(m)[0])
            return tuple(nxt)

        tpad = jnp.full((L,), t0, _f32)
        woff = z32
        for j in range(NSEG):
            oj = cp_offs[j]
            cand[pl.ds(j * SEG + oj, L)] = tpad
            ncj = (oj + (L - 1)) >> 4

            def pk_body(k, w, j=j):
                packed[pl.ds(w * L, L)] = cand[pl.ds(j * SEG + k * L, L)]
                return w + 1

            woff = lax.fori_loop(0, ncj, pk_body, woff)
        packed[pl.ds(woff * L, L)] = tpad
        trips = (woff + 1) >> 1

        def bis_body(_, lohi):
            lo, hi = lohi
            mid = _f32(0.5) * (lo + hi)

            def s_body(k, accs):
                s0, s1 = accs
                v0 = packed[pl.ds((2 * k) * L, L)]
                v1 = packed[pl.ds((2 * k + 1) * L, L)]
                return (s0 + jnp.maximum(v0 - mid, _f32(0.0)),
                        s1 + jnp.maximum(v1 - mid, _f32(0.0)))

            zv = jnp.zeros((L,), _f32)
            s0, s1 = lax.fori_loop(0, trips, s_body, (zv, zv))
            s = jnp.sum(s0 + s1)
            ge = s >= _f32(1.0)
            return (jnp.where(ge, mid, lo), jnp.where(ge, hi, mid))

        lo, hi = lax.fori_loop(0, BISECT_ITERS, bis_body, (t0, rowmax))

        def fin_body(k, carry):
            sacc, cacc = carry
            v0 = packed[pl.ds((2 * k) * L, L)]
            v1 = packed[pl.ds((2 * k + 1) * L, L)]
            m0 = v0 > lo
            m1 = v1 > lo
            return (sacc + jnp.where(m0, v0, _f32(0.0))
                    + jnp.where(m1, v1, _f32(0.0)),
                    cacc + m0.astype(_f32) + m1.astype(_f32))

        sacc, cacc = lax.fori_loop(
            0, trips, fin_body,
            (jnp.zeros((L,), _f32), jnp.zeros((L,), _f32)))
        sv = jnp.full((L,), jnp.sum(sacc) - _f32(1.0), _f32)
        cv = jnp.full((L,), jnp.maximum(jnp.sum(cacc), _f32(1.0)), _f32)
        tau = sv / cv

        @plsc.parallel_loop(0, NCHUNK, step=4, unroll=2)
        def _out(i):
            for j in range(4):
                sl = pl.ds((i + j) * L, L)
                xv[r, sl] = jnp.maximum(xv[r, sl] - tau, _f32(0.0))

        if r == 0:
            out0 = pltpu.async_copy(xv.at[pl.ds(0, 1)],
                                    out_hbm.at[pl.ds(base, 1)], sem_out0)

    pltpu.sync_copy(xv.at[pl.ds(1, 1)], out_hbm.at[pl.ds(base + 1, 1)])
    out0.wait()


def kernel(input):
    return _sparsemax_sc(input)
